# baseline (speedup 1.0000x reference)
import jax
import jax.numpy as jnp
from jax.experimental import pallas as pl


def _copy_block(t_ref, o_ref):
    o_ref[...] = t_ref[...]


def kernel(x, table):
    seq = x.shape[1]
    emb = table.shape[1]
    block = 4096
    out = pl.pallas_call(
        _copy_block,
        grid=(seq // block,),
        in_specs=[pl.BlockSpec((block, emb), lambda i: (i, 0))],
        out_specs=pl.BlockSpec((block, emb), lambda i: (i, 0)),
        out_shape=jax.ShapeDtypeStruct((seq, emb), table.dtype),
    )(table)
    return out[None, :, :]
